# baseline (speedup 1.0000x reference)
import functools

import jax
import jax.numpy as jnp
from jax import lax
from jax.experimental import pallas as pl
from jax.experimental.pallas import tpu as pltpu
from jax.experimental.pallas import tpu_sc as plsc

EPS = 1e-12

NUM_SUBCORES = 16
GATHER_CHUNK = 64
NSPLIT = 2


def _sc_gather_half(table, idx):
    b, = idx.shape
    _, d = table.shape
    b_per_w = b // NUM_SUBCORES
    n_chunks = b_per_w // GATHER_CHUNK
    mesh = plsc.VectorSubcoreMesh(
        core_axis_name="c", subcore_axis_name="s", num_cores=1
    )

    @functools.partial(
        pl.kernel,
        mesh=mesh,
        out_type=jax.ShapeDtypeStruct((b, d), jnp.float32),
        scratch_types=[
            pltpu.VMEM((b_per_w,), jnp.int32),
            pltpu.VMEM((GATHER_CHUNK, d), jnp.float32),
            pltpu.VMEM((GATHER_CHUNK, d), jnp.float32),
            pltpu.SemaphoreType.DMA,
            pltpu.SemaphoreType.DMA,
            pltpu.SemaphoreType.DMA,
        ],
    )
    def gather_kernel(table_hbm, idx_hbm, out_hbm, idx_v, rows_a, rows_b,
                      sem_g, sem_wa, sem_wb):
        wid = lax.axis_index("s")
        base = wid * b_per_w
        pltpu.sync_copy(idx_hbm.at[pl.ds(base, b_per_w)], idx_v)

        bufs = (rows_a, rows_b)
        wsems = (sem_wa, sem_wb)
        for i in range(n_chunks):
            buf, wsem = bufs[i % 2], wsems[i % 2]
            if i >= 2:
                pltpu.make_async_copy(buf, out_hbm.at[pl.ds(0, GATHER_CHUNK)],
                                      wsem).wait()
            c = i * GATHER_CHUNK
            pltpu.async_copy(
                table_hbm.at[idx_v.at[pl.ds(c, GATHER_CHUNK)]], buf, sem_g
            ).wait()
            pltpu.async_copy(buf, out_hbm.at[pl.ds(base + c, GATHER_CHUNK)], wsem)
        for i in range(min(2, n_chunks)):
            pltpu.make_async_copy(
                bufs[i % 2], out_hbm.at[pl.ds(0, GATHER_CHUNK)], wsems[i % 2]
            ).wait()

    return gather_kernel(table, idx)


def _ln_body(x_ref, pos_ref, gamma_ref, beta_ref, *rest):
    out_ref = rest[-1]
    pos_start = (pl.program_id(0) % (pos_ref.shape[0] // x_ref.shape[0])) \
        * x_ref.shape[0]
    x = x_ref[...] + pos_ref[pl.ds(pos_start, x_ref.shape[0]), :]
    mean = jnp.mean(x, axis=-1, keepdims=True)
    centered = x - mean
    var = jnp.mean(centered * centered, axis=-1, keepdims=True)
    normed = centered * lax.rsqrt(var + EPS)
    out_ref[...] = normed * gamma_ref[...] + beta_ref[...]


def _tc_add_ln_half(gathered, pos_table, gamma, beta, half_idx, n_total,
                    block_rows, prev_out):
    rows, d = gathered.shape
    s = pos_table.shape[0]
    blocks_per_half = rows // block_rows
    in_specs = [
        pl.BlockSpec((block_rows, d), lambda i: (i, 0)),
        pl.BlockSpec((s, d), lambda i: (0, 0)),
        pl.BlockSpec((1, d), lambda i: (0, 0)),
        pl.BlockSpec((1, d), lambda i: (0, 0)),
    ]
    operands = [gathered, pos_table, gamma.reshape(1, d), beta.reshape(1, d)]
    aliases = {}
    if prev_out is not None:
        in_specs.append(pl.BlockSpec(memory_space=pl.ANY))
        operands.append(prev_out)
        aliases = {4: 0}
    base = half_idx * blocks_per_half
    return pl.pallas_call(
        _ln_body,
        grid=(blocks_per_half,),
        in_specs=in_specs,
        out_specs=pl.BlockSpec((block_rows, d), lambda i: (base + i, 0)),
        out_shape=jax.ShapeDtypeStruct((n_total, d), jnp.float32),
        input_output_aliases=aliases,
        compiler_params=pltpu.CompilerParams(
            dimension_semantics=("arbitrary",),
        ),
    )(*operands)


def kernel(input_ids, word_table, pos_table, gamma, beta):
    batch, seq = input_ids.shape
    d = word_table.shape[1]
    n = batch * seq
    ids_flat = input_ids.reshape(-1).astype(jnp.int32)
    rows_per_split = n // NSPLIT
    gathered = [
        _sc_gather_half(
            word_table,
            lax.slice(ids_flat, (h * rows_per_split,),
                      ((h + 1) * rows_per_split,)),
        )
        for h in range(NSPLIT)
    ]
    out = None
    for h in range(NSPLIT):
        out = _tc_add_ln_half(gathered[h], pos_table, gamma, beta, h, n,
                              block_rows=1024, prev_out=out)
    return out.reshape(batch, seq, d)


# speedup vs baseline: 1.1359x; 1.1359x over previous
import functools

import jax
import jax.numpy as jnp
from jax import lax
from jax.experimental import pallas as pl
from jax.experimental.pallas import tpu as pltpu
from jax.experimental.pallas import tpu_sc as plsc

EPS = 1e-12

NUM_CORES = 2
NUM_SUBCORES = 16
GATHER_CHUNK = 64


def _sc_gather(table, idx):
    b, = idx.shape
    _, d = table.shape
    n_workers = NUM_CORES * NUM_SUBCORES
    b_per_w = b // n_workers
    n_chunks = b_per_w // GATHER_CHUNK
    mesh = plsc.VectorSubcoreMesh(
        core_axis_name="c", subcore_axis_name="s", num_cores=NUM_CORES
    )

    @functools.partial(
        pl.kernel,
        mesh=mesh,
        out_type=jax.ShapeDtypeStruct((b, d), jnp.float32),
        scratch_types=[
            pltpu.VMEM((b_per_w,), jnp.int32),
            pltpu.VMEM((GATHER_CHUNK, d), jnp.float32),
            pltpu.VMEM((GATHER_CHUNK, d), jnp.float32),
            pltpu.SemaphoreType.DMA,
            pltpu.SemaphoreType.DMA,
            pltpu.SemaphoreType.DMA,
        ],
    )
    def gather_kernel(table_hbm, idx_hbm, out_hbm, idx_v, rows_a, rows_b,
                      sem_g, sem_wa, sem_wb):
        wid = lax.axis_index("c") * NUM_SUBCORES + lax.axis_index("s")
        base = wid * b_per_w
        pltpu.sync_copy(idx_hbm.at[pl.ds(base, b_per_w)], idx_v)

        bufs = (rows_a, rows_b)
        wsems = (sem_wa, sem_wb)
        for i in range(n_chunks):
            buf, wsem = bufs[i % 2], wsems[i % 2]
            if i >= 2:
                pltpu.make_async_copy(buf, out_hbm.at[pl.ds(0, GATHER_CHUNK)],
                                      wsem).wait()
            c = i * GATHER_CHUNK
            pltpu.async_copy(
                table_hbm.at[idx_v.at[pl.ds(c, GATHER_CHUNK)]], buf, sem_g
            ).wait()
            pltpu.async_copy(buf, out_hbm.at[pl.ds(base + c, GATHER_CHUNK)], wsem)
        for i in range(min(2, n_chunks)):
            pltpu.make_async_copy(
                bufs[i % 2], out_hbm.at[pl.ds(0, GATHER_CHUNK)], wsems[i % 2]
            ).wait()

    return gather_kernel(table, idx)


def _ln_body(x_ref, pos_ref, gamma_ref, beta_ref, out_ref):
    pos_start = (pl.program_id(0) % (pos_ref.shape[0] // x_ref.shape[0])) \
        * x_ref.shape[0]
    x = x_ref[...] + pos_ref[pl.ds(pos_start, x_ref.shape[0]), :]
    mean = jnp.mean(x, axis=-1, keepdims=True)
    centered = x - mean
    var = jnp.mean(centered * centered, axis=-1, keepdims=True)
    normed = centered * lax.rsqrt(var + EPS)
    out_ref[...] = normed * gamma_ref[...] + beta_ref[...]


def _tc_add_ln(gathered, pos_table, gamma, beta, block_rows):
    rows, d = gathered.shape
    s = pos_table.shape[0]
    n_blocks = rows // block_rows
    return pl.pallas_call(
        _ln_body,
        grid=(n_blocks,),
        in_specs=[
            pl.BlockSpec((block_rows, d), lambda i: (i, 0)),
            pl.BlockSpec((s, d), lambda i: (0, 0)),
            pl.BlockSpec((1, d), lambda i: (0, 0)),
            pl.BlockSpec((1, d), lambda i: (0, 0)),
        ],
        out_specs=pl.BlockSpec((block_rows, d), lambda i: (i, 0)),
        out_shape=jax.ShapeDtypeStruct((rows, d), jnp.float32),
        compiler_params=pltpu.CompilerParams(
            dimension_semantics=("arbitrary",),
        ),
    )(gathered, pos_table, gamma.reshape(1, d), beta.reshape(1, d))


def kernel(input_ids, word_table, pos_table, gamma, beta):
    batch, seq = input_ids.shape
    d = word_table.shape[1]
    n = batch * seq
    ids_flat = input_ids.reshape(-1).astype(jnp.int32)
    gathered = _sc_gather(word_table, ids_flat)
    out = _tc_add_ln(gathered, pos_table, gamma, beta, block_rows=1024)
    return out.reshape(batch, seq, d)
